# baseline (device time: 252028 ns/iter reference)
import jax
import jax.numpy as jnp
from jax import lax
from jax.experimental import pallas as pl
from jax.experimental.pallas import tpu as pltpu

T = 2048
D = 1024
CH = 128
NCH = T // CH


def kernel(x, dest):
    my_y = lax.axis_index("y")

    n0 = jnp.sum((dest == 0).astype(jnp.int32))
    n_keep = jnp.where(my_y == 0, n0, T - n0).astype(jnp.int32)
    m = (T - n_keep).astype(jnp.int32)
    keep_off = my_y * m
    recv_dst = (1 - my_y) * n_keep
    nch = (m + CH - 1) // CH

    order2 = jnp.argsort(dest != my_y, stable=True).astype(jnp.int32)
    j = jnp.arange(T, dtype=jnp.int32)
    g_send = order2[jnp.clip(n_keep + j, 0, T - 1)]
    g_out = order2[jnp.clip(j - keep_off, 0, T - 1)]

    x_bf = x.astype(jnp.bfloat16)
    meta = jnp.stack([n_keep, m, keep_off, recv_dst, nch]).astype(jnp.int32)

    def body(meta_ref, x_ref, gs_ref, go_ref, out_ref, sbuf_ref, recv_ref,
             send_sems, recv_sems):
        n_keep_ = meta_ref[0]
        m_ = meta_ref[1]
        keep_off_ = meta_ref[2]
        recv_dst_ = meta_ref[3]
        nch_ = meta_ref[4]

        ax = lax.axis_index("x")
        ay = lax.axis_index("y")
        az = lax.axis_index("z")
        peer = (ax, 1 - ay, az)

        barrier = pltpu.get_barrier_semaphore()
        pl.semaphore_signal(
            barrier, inc=1, device_id=peer, device_id_type=pl.DeviceIdType.MESH
        )
        pl.semaphore_wait(barrier, 1)

        col = lax.broadcasted_iota(jnp.int32, (CH, T), 1)

        def permute_chunk(g_ref, c):
            gc = g_ref[c * CH:(c + 1) * CH, :]
            p = (gc == col).astype(jnp.bfloat16)
            acc = jnp.dot(p, x_ref[...], preferred_element_type=jnp.float32)
            return acc.astype(jnp.bfloat16)

        def cstart(i):
            tail = jnp.maximum(0, ((m_ + 7) // 8) * 8 - CH)
            s = jnp.where(i == nch_ - 1, tail, i * CH)
            return pl.multiple_of(s, 8)

        def mk_chunk(i):
            s = cstart(i)
            return pltpu.make_async_remote_copy(
                src_ref=sbuf_ref.at[pl.ds(s, CH), :],
                dst_ref=recv_ref.at[pl.ds(s, CH), :],
                send_sem=send_sems.at[i],
                recv_sem=recv_sems.at[i],
                device_id=peer,
                device_id_type=pl.DeviceIdType.MESH,
            )

        for i in range(NCH):
            rdma = mk_chunk(i)

            @pl.when(i < nch_)
            def _(i=i, rdma=rdma):
                sbuf_ref[i * CH:(i + 1) * CH, :] = permute_chunk(gs_ref, i)
                rdma.start()

        for c in range(NCH):
            overlaps = ((c + 1) * CH > keep_off_) & (c * CH < keep_off_ + n_keep_)

            @pl.when(overlaps)
            def _(c=c):
                out_ref[c * CH:(c + 1) * CH, :] = permute_chunk(go_ref, c)

        for i in range(NCH):
            rdma = mk_chunk(i)

            @pl.when(i < nch_)
            def _(rdma=rdma):
                rdma.wait_send()
                rdma.wait_recv()

        row_ids = lax.broadcasted_iota(jnp.int32, (T, 1), 0)
        mine = (row_ids >= keep_off_) & (row_ids < keep_off_ + n_keep_)
        out_ref[...] = jnp.where(
            mine, out_ref[...], pltpu.roll(recv_ref[...], recv_dst_, 0)
        )

    return pl.pallas_call(
        body,
        out_shape=jax.ShapeDtypeStruct((T, D), jnp.bfloat16),
        in_specs=[
            pl.BlockSpec(memory_space=pltpu.SMEM),
            pl.BlockSpec(memory_space=pltpu.VMEM),
            pl.BlockSpec(memory_space=pltpu.VMEM),
            pl.BlockSpec(memory_space=pltpu.VMEM),
        ],
        out_specs=pl.BlockSpec(memory_space=pltpu.VMEM),
        scratch_shapes=[
            pltpu.VMEM((T, D), jnp.bfloat16),
            pltpu.VMEM((T, D), jnp.bfloat16),
            pltpu.SemaphoreType.DMA((NCH,)),
            pltpu.SemaphoreType.DMA((NCH,)),
        ],
        compiler_params=pltpu.CompilerParams(collective_id=0),
    )(meta, x_bf, g_send.reshape(T, 1), g_out.reshape(T, 1))


# device time: 45874 ns/iter; 5.4939x vs baseline; 5.4939x over previous
import jax
import jax.numpy as jnp
from jax import lax
from jax.experimental import pallas as pl
from jax.experimental.pallas import tpu as pltpu

T = 2048
D = 1024
CH = 128
NCH = T // CH


def kernel(x, dest):
    my_y = lax.axis_index("y")

    n0 = jnp.sum((dest == 0).astype(jnp.int32))
    n_keep = jnp.where(my_y == 0, n0, T - n0).astype(jnp.int32)
    m = (T - n_keep).astype(jnp.int32)
    keep_off = my_y * m
    recv_dst = (1 - my_y) * n_keep
    nch = (m + CH - 1) // CH

    order2 = jnp.argsort(dest != my_y, stable=True).astype(jnp.int32)
    g_send = jnp.roll(order2, -n_keep)
    g_out = jnp.roll(order2, keep_off)

    x_bf = x.astype(jnp.bfloat16)
    meta = jnp.stack([n_keep, m, keep_off, recv_dst, nch]).astype(jnp.int32)

    def body(meta_ref, x_ref, gs_ref, go_ref, out_ref, sbuf_ref, recv_ref,
             send_sems, recv_sems):
        n_keep_ = meta_ref[0]
        m_ = meta_ref[1]
        keep_off_ = meta_ref[2]
        recv_dst_ = meta_ref[3]
        nch_ = meta_ref[4]

        ax = lax.axis_index("x")
        ay = lax.axis_index("y")
        az = lax.axis_index("z")
        peer = (ax, 1 - ay, az)

        barrier = pltpu.get_barrier_semaphore()
        pl.semaphore_signal(
            barrier, inc=1, device_id=peer, device_id_type=pl.DeviceIdType.MESH
        )
        pl.semaphore_wait(barrier, 1)

        col = lax.broadcasted_iota(jnp.int32, (CH, T), 1)

        def permute_chunk(g_ref, c):
            gc = g_ref[c * CH:(c + 1) * CH, :]
            p = (gc == col).astype(jnp.bfloat16)
            acc = jnp.dot(p, x_ref[...], preferred_element_type=jnp.float32)
            return acc.astype(jnp.bfloat16)

        def cstart(i):
            tail = jnp.maximum(0, ((m_ + 7) // 8) * 8 - CH)
            s = jnp.where(i == nch_ - 1, tail, i * CH)
            return pl.multiple_of(s, 8)

        def mk_chunk(i):
            s = cstart(i)
            return pltpu.make_async_remote_copy(
                src_ref=sbuf_ref.at[pl.ds(s, CH), :],
                dst_ref=recv_ref.at[pl.ds(s, CH), :],
                send_sem=send_sems.at[i],
                recv_sem=recv_sems.at[i],
                device_id=peer,
                device_id_type=pl.DeviceIdType.MESH,
            )

        for i in range(NCH):
            rdma = mk_chunk(i)

            @pl.when(i < nch_)
            def _(i=i, rdma=rdma):
                sbuf_ref[i * CH:(i + 1) * CH, :] = permute_chunk(gs_ref, i)
                rdma.start()

        for c in range(NCH):
            overlaps = ((c + 1) * CH > keep_off_) & (c * CH < keep_off_ + n_keep_)

            @pl.when(overlaps)
            def _(c=c):
                out_ref[c * CH:(c + 1) * CH, :] = permute_chunk(go_ref, c)

        for i in range(NCH):
            rdma = mk_chunk(i)

            @pl.when(i < nch_)
            def _(rdma=rdma):
                rdma.wait_send()
                rdma.wait_recv()

        row_ids = lax.broadcasted_iota(jnp.int32, (T, 1), 0)
        mine = (row_ids >= keep_off_) & (row_ids < keep_off_ + n_keep_)
        out_ref[...] = jnp.where(
            mine, out_ref[...], pltpu.roll(recv_ref[...], recv_dst_, 0)
        )

    return pl.pallas_call(
        body,
        out_shape=jax.ShapeDtypeStruct((T, D), jnp.bfloat16),
        in_specs=[
            pl.BlockSpec(memory_space=pltpu.SMEM),
            pl.BlockSpec(memory_space=pltpu.VMEM),
            pl.BlockSpec(memory_space=pltpu.VMEM),
            pl.BlockSpec(memory_space=pltpu.VMEM),
        ],
        out_specs=pl.BlockSpec(memory_space=pltpu.VMEM),
        scratch_shapes=[
            pltpu.VMEM((T, D), jnp.bfloat16),
            pltpu.VMEM((T, D), jnp.bfloat16),
            pltpu.SemaphoreType.DMA((NCH,)),
            pltpu.SemaphoreType.DMA((NCH,)),
        ],
        compiler_params=pltpu.CompilerParams(collective_id=0),
    )(meta, x_bf, g_send.reshape(T, 1), g_out.reshape(T, 1))


# device time: 40373 ns/iter; 6.2425x vs baseline; 1.1363x over previous
import jax
import jax.numpy as jnp
from jax import lax
from jax.experimental import pallas as pl
from jax.experimental.pallas import tpu as pltpu

T = 2048
D = 1024
CH = 128
NCH = T // CH


def kernel(x, dest):
    my_y = lax.axis_index("y")

    keep = (dest == my_y).astype(jnp.int32)
    kcum = jnp.cumsum(keep)
    n_keep = kcum[T - 1]
    m = (T - n_keep).astype(jnp.int32)
    keep_off = my_y * m
    recv_dst = (1 - my_y) * n_keep
    nch = (m + CH - 1) // CH

    scum = jnp.arange(1, T + 1, dtype=jnp.int32) - kcum
    d_out = jnp.where(keep == 1, kcum - 1 + keep_off, -1).astype(jnp.int32)
    d_send = jnp.where(keep == 1, -1, scum - 1).astype(jnp.int32)

    x_bf = x.astype(jnp.bfloat16)
    meta = jnp.stack([n_keep, m, keep_off, recv_dst, nch]).astype(jnp.int32)

    def body(meta_ref, x_ref, ds_ref, do_ref, out_ref, sbuf_ref, recv_ref,
             send_sems, recv_sems):
        n_keep_ = meta_ref[0]
        m_ = meta_ref[1]
        keep_off_ = meta_ref[2]
        recv_dst_ = meta_ref[3]
        nch_ = meta_ref[4]

        ax = lax.axis_index("x")
        ay = lax.axis_index("y")
        az = lax.axis_index("z")
        peer = (ax, 1 - ay, az)

        barrier = pltpu.get_barrier_semaphore()
        pl.semaphore_signal(
            barrier, inc=1, device_id=peer, device_id_type=pl.DeviceIdType.MESH
        )
        pl.semaphore_wait(barrier, 1)

        row = lax.broadcasted_iota(jnp.int32, (CH, T), 0)

        def permute_chunk(d_ref, c):
            p = ((row + c * CH) == d_ref[...]).astype(jnp.bfloat16)
            acc = jnp.dot(p, x_ref[...], preferred_element_type=jnp.float32)
            return acc.astype(jnp.bfloat16)

        def cstart(i):
            tail = jnp.maximum(0, ((m_ + 7) // 8) * 8 - CH)
            s = jnp.where(i == nch_ - 1, tail, i * CH)
            return pl.multiple_of(s, 8)

        def mk_chunk(i):
            s = cstart(i)
            return pltpu.make_async_remote_copy(
                src_ref=sbuf_ref.at[pl.ds(s, CH), :],
                dst_ref=recv_ref.at[pl.ds(s, CH), :],
                send_sem=send_sems.at[i],
                recv_sem=recv_sems.at[i],
                device_id=peer,
                device_id_type=pl.DeviceIdType.MESH,
            )

        for i in range(NCH):
            rdma = mk_chunk(i)

            @pl.when(i < nch_)
            def _(i=i, rdma=rdma):
                sbuf_ref[i * CH:(i + 1) * CH, :] = permute_chunk(ds_ref, i)
                rdma.start()

        for c in range(NCH):
            overlaps = ((c + 1) * CH > keep_off_) & (c * CH < keep_off_ + n_keep_)

            @pl.when(overlaps)
            def _(c=c):
                out_ref[c * CH:(c + 1) * CH, :] = permute_chunk(do_ref, c)

        for i in range(NCH):
            rdma = mk_chunk(i)

            @pl.when(i < nch_)
            def _(rdma=rdma):
                rdma.wait_send()
                rdma.wait_recv()

        row_ids = lax.broadcasted_iota(jnp.int32, (T, 1), 0)
        mine = (row_ids >= keep_off_) & (row_ids < keep_off_ + n_keep_)
        out_ref[...] = jnp.where(
            mine, out_ref[...], pltpu.roll(recv_ref[...], recv_dst_, 0)
        )

    return pl.pallas_call(
        body,
        out_shape=jax.ShapeDtypeStruct((T, D), jnp.bfloat16),
        in_specs=[
            pl.BlockSpec(memory_space=pltpu.SMEM),
            pl.BlockSpec(memory_space=pltpu.VMEM),
            pl.BlockSpec(memory_space=pltpu.VMEM),
            pl.BlockSpec(memory_space=pltpu.VMEM),
        ],
        out_specs=pl.BlockSpec(memory_space=pltpu.VMEM),
        scratch_shapes=[
            pltpu.VMEM((T, D), jnp.bfloat16),
            pltpu.VMEM((T, D), jnp.bfloat16),
            pltpu.SemaphoreType.DMA((NCH,)),
            pltpu.SemaphoreType.DMA((NCH,)),
        ],
        compiler_params=pltpu.CompilerParams(collective_id=0),
    )(meta, x_bf, d_send.reshape(1, T), d_out.reshape(1, T))


# device time: 36088 ns/iter; 6.9837x vs baseline; 1.1187x over previous
import jax
import jax.numpy as jnp
from jax import lax
from jax.experimental import pallas as pl
from jax.experimental.pallas import tpu as pltpu

T = 2048
D = 1024
CH = 128
NCH = T // CH


def kernel(x, dest):
    my_y = lax.axis_index("y")

    keep = (dest == my_y).astype(jnp.int32)
    kcum = jnp.cumsum(keep)
    n_keep = kcum[T - 1]
    m = (T - n_keep).astype(jnp.int32)
    keep_off = my_y * m
    rd_send = my_y * n_keep
    rd_recv = (1 - my_y) * n_keep

    scum = jnp.arange(1, T + 1, dtype=jnp.int32) - kcum
    d_out = jnp.where(keep == 1, kcum - 1 + keep_off, -1).astype(jnp.int32)
    d_send = jnp.where(keep == 1, -1, scum - 1 + rd_send).astype(jnp.int32)

    x_bf = x.astype(jnp.bfloat16)
    meta = jnp.stack([n_keep, m, keep_off, rd_send, rd_recv]).astype(jnp.int32)

    def body(meta_ref, x_ref, ds_ref, do_ref, out_ref, sbuf_ref, recv_ref,
             send_sems, recv_sems):
        n_keep_ = meta_ref[0]
        m_ = meta_ref[1]
        keep_off_ = meta_ref[2]
        rd_send_ = meta_ref[3]
        rd_recv_ = meta_ref[4]

        ax = lax.axis_index("x")
        ay = lax.axis_index("y")
        az = lax.axis_index("z")
        peer = (ax, 1 - ay, az)

        barrier = pltpu.get_barrier_semaphore()
        pl.semaphore_signal(
            barrier, inc=1, device_id=peer, device_id_type=pl.DeviceIdType.MESH
        )
        pl.semaphore_wait(barrier, 1)

        def overlaps(c, off, n):
            return ((c + 1) * CH > off) & (c * CH < off + n)

        row = lax.broadcasted_iota(jnp.int32, (CH, T), 0)

        def permute_chunk(d_ref, c):
            p = ((row + c * CH) == d_ref[...]).astype(jnp.bfloat16)
            acc = jnp.dot(p, x_ref[...], preferred_element_type=jnp.float32)
            return acc.astype(jnp.bfloat16)

        def mk_chunk(c):
            return pltpu.make_async_remote_copy(
                src_ref=sbuf_ref.at[pl.ds(c * CH, CH), :],
                dst_ref=recv_ref.at[pl.ds(c * CH, CH), :],
                send_sem=send_sems.at[c],
                recv_sem=recv_sems.at[c],
                device_id=peer,
                device_id_type=pl.DeviceIdType.MESH,
            )

        for c in range(NCH):
            rdma = mk_chunk(c)

            @pl.when(overlaps(c, rd_send_, m_))
            def _(c=c, rdma=rdma):
                sbuf_ref[c * CH:(c + 1) * CH, :] = permute_chunk(ds_ref, c)
                rdma.start()

        for c in range(NCH):
            has_keep = overlaps(c, keep_off_, n_keep_)

            @pl.when(has_keep)
            def _(c=c):
                out_ref[c * CH:(c + 1) * CH, :] = permute_chunk(do_ref, c)

            @pl.when(jnp.logical_not(has_keep))
            def _(c=c):
                out_ref[c * CH:(c + 1) * CH, :] = jnp.zeros(
                    (CH, D), jnp.bfloat16
                )

        for c in range(NCH):
            rdma = mk_chunk(c)

            @pl.when(overlaps(c, rd_recv_, m_))
            def _(c=c, rdma=rdma):
                rdma.wait_recv()
                out_ref[c * CH:(c + 1) * CH, :] = (
                    out_ref[c * CH:(c + 1) * CH, :]
                    + recv_ref[c * CH:(c + 1) * CH, :]
                )

            @pl.when(overlaps(c, rd_send_, m_))
            def _(rdma=rdma):
                rdma.wait_send()

    return pl.pallas_call(
        body,
        out_shape=jax.ShapeDtypeStruct((T, D), jnp.bfloat16),
        in_specs=[
            pl.BlockSpec(memory_space=pltpu.SMEM),
            pl.BlockSpec(memory_space=pltpu.VMEM),
            pl.BlockSpec(memory_space=pltpu.VMEM),
            pl.BlockSpec(memory_space=pltpu.VMEM),
        ],
        out_specs=pl.BlockSpec(memory_space=pltpu.VMEM),
        scratch_shapes=[
            pltpu.VMEM((T, D), jnp.bfloat16),
            pltpu.VMEM((T, D), jnp.bfloat16),
            pltpu.SemaphoreType.DMA((NCH,)),
            pltpu.SemaphoreType.DMA((NCH,)),
        ],
        compiler_params=pltpu.CompilerParams(collective_id=0),
    )(meta, x_bf, d_send.reshape(1, T), d_out.reshape(1, T))


# device time: 33915 ns/iter; 7.4312x vs baseline; 1.0641x over previous
import jax
import jax.numpy as jnp
from jax import lax
from jax.experimental import pallas as pl
from jax.experimental.pallas import tpu as pltpu

T = 2048
D = 1024
CH = 128
NCH = T // CH


def kernel(x, dest):
    x_bf = x.astype(jnp.bfloat16)

    def body(x_ref, dest_ref, out_ref, sbuf_ref, recv_ref,
             send_sems, recv_sems):
        ax = lax.axis_index("x")
        ay = lax.axis_index("y")
        az = lax.axis_index("z")
        peer = (ax, 1 - ay, az)

        barrier = pltpu.get_barrier_semaphore()
        pl.semaphore_signal(
            barrier, inc=1, device_id=peer, device_id_type=pl.DeviceIdType.MESH
        )

        lane = lax.broadcasted_iota(jnp.int32, (1, T), 1)
        keep = (dest_ref[...] == ay).astype(jnp.int32)

        kcum = keep
        s = 1
        while s < T:
            kcum = kcum + jnp.where(
                lane >= s, pltpu.roll(kcum, s, 1), 0
            )
            s *= 2

        n_keep = jnp.sum(keep)
        m = T - n_keep
        keep_off = ay * m
        rd_send = ay * n_keep
        rd_recv = (1 - ay) * n_keep

        scum = (lane + 1) - kcum
        d_out = jnp.where(keep == 1, kcum - 1 + keep_off, -1)
        d_send = jnp.where(keep == 1, -1, scum - 1 + rd_send)

        pl.semaphore_wait(barrier, 1)

        def overlaps(c, off, n):
            return ((c + 1) * CH > off) & (c * CH < off + n)

        row = lax.broadcasted_iota(jnp.int32, (CH, T), 0)

        def permute_chunk(d, c):
            p = ((row + c * CH) == d).astype(jnp.bfloat16)
            acc = jnp.dot(p, x_ref[...], preferred_element_type=jnp.float32)
            return acc.astype(jnp.bfloat16)

        def mk_chunk(c):
            return pltpu.make_async_remote_copy(
                src_ref=sbuf_ref.at[pl.ds(c * CH, CH), :],
                dst_ref=recv_ref.at[pl.ds(c * CH, CH), :],
                send_sem=send_sems.at[c],
                recv_sem=recv_sems.at[c],
                device_id=peer,
                device_id_type=pl.DeviceIdType.MESH,
            )

        for c in range(NCH):
            rdma = mk_chunk(c)

            @pl.when(overlaps(c, rd_send, m))
            def _(c=c, rdma=rdma):
                sbuf_ref[c * CH:(c + 1) * CH, :] = permute_chunk(d_send, c)
                rdma.start()

        for c in range(NCH):
            has_keep = overlaps(c, keep_off, n_keep)

            @pl.when(has_keep)
            def _(c=c):
                out_ref[c * CH:(c + 1) * CH, :] = permute_chunk(d_out, c)

            @pl.when(jnp.logical_not(has_keep))
            def _(c=c):
                out_ref[c * CH:(c + 1) * CH, :] = jnp.zeros(
                    (CH, D), jnp.bfloat16
                )

        for c in range(NCH):
            rdma = mk_chunk(c)

            @pl.when(overlaps(c, rd_recv, m))
            def _(c=c, rdma=rdma):
                rdma.wait_recv()
                out_ref[c * CH:(c + 1) * CH, :] = (
                    out_ref[c * CH:(c + 1) * CH, :]
                    + recv_ref[c * CH:(c + 1) * CH, :]
                )

            @pl.when(overlaps(c, rd_send, m))
            def _(rdma=rdma):
                rdma.wait_send()

    return pl.pallas_call(
        body,
        out_shape=jax.ShapeDtypeStruct((T, D), jnp.bfloat16),
        in_specs=[
            pl.BlockSpec(memory_space=pltpu.VMEM),
            pl.BlockSpec(memory_space=pltpu.VMEM),
        ],
        out_specs=pl.BlockSpec(memory_space=pltpu.VMEM),
        scratch_shapes=[
            pltpu.VMEM((T, D), jnp.bfloat16),
            pltpu.VMEM((T, D), jnp.bfloat16),
            pltpu.SemaphoreType.DMA((NCH,)),
            pltpu.SemaphoreType.DMA((NCH,)),
        ],
        compiler_params=pltpu.CompilerParams(collective_id=0),
    )(x_bf, dest.reshape(1, T))
